# baseline (speedup 1.0000x reference)
import functools

import jax
import jax.numpy as jnp
import numpy as np
from jax import lax
from jax.experimental import pallas as pl
from jax.experimental.pallas import tpu as pltpu
from jax.experimental.pallas import tpu_sc as plsc

F_FIELDS = 26
VOCAB = 100000
EMB_D = 32
BATCH = 16384
HIDDEN = 1024
OUT_D = 64

_info = plsc.get_sparse_core_info()
_NC, _NS = _info.num_cores, _info.num_subcores
_NW = _NC * _NS

_NJ = 7
_B_PER_W = BATCH // _NW
_SRC_PER_W = _B_PER_W * F_FIELDS
_CHUNK = 1024
_NCHUNK = _SRC_PER_W // _CHUNK
_SCAT = _CHUNK // 128
_OUT_ROWS = _NJ * 4 * BATCH

_TR_Q = 4096
_TR_QS = 12
_TR_CW = 4 * _TR_Q
_TR_GRID = -(-(F_FIELDS * VOCAB) // _TR_CW)
_TBL_LINES = _TR_GRID * _TR_Q


def _tr_body(xt_ref, o_ref):
    x = xt_ref[...]
    z = jnp.concatenate(
        [x[:, u * _TR_Q:(u + 1) * _TR_Q] for u in range(4)], axis=0
    )
    o_ref[...] = jnp.swapaxes(z, 0, 1).astype(jnp.bfloat16)


def _transpose_table(tables_t):
    return pl.pallas_call(
        _tr_body,
        grid=(_TR_GRID,),
        in_specs=[pl.BlockSpec((EMB_D, _TR_CW), lambda i: (0, i))],
        out_specs=pl.BlockSpec((_TR_Q, 128), lambda i: (i, 0)),
        out_shape=jax.ShapeDtypeStruct((_TBL_LINES, 128), jnp.bfloat16),
    )(tables_t)


_P = np.arange(_SRC_PER_W)
_F = _P % F_FIELDS
_OFF_TBL = (_F * VOCAB).astype(np.int32)
_J = np.minimum(_F // 4, _NJ - 1)
_T = _F - 4 * _J
_DST_TBL = np.empty((_NW, _SRC_PER_W), np.int32)
for _w in range(_NW):
    _B = _w * _B_PER_W + _P // F_FIELDS
    _DST_TBL[_w] = _J * (4 * BATCH) + 4 * _B + _T
_DST_TBL = _DST_TBL.reshape(_NW, _NCHUNK * _SCAT, 128)


def _sc_gather_body(off_hbm, dst_hbm, uidx_hbm, table_hbm, out_hbm,
                    off_v, dst_v, uidx_v, gl0, gl1, rows0, rows1,
                    gsem0, gsem1, ssem0, ssem1):
    wid = lax.axis_index("s") * _NC + lax.axis_index("c")
    u0 = pl.multiple_of(wid * _SRC_PER_W, 8)
    pltpu.sync_copy(off_hbm, off_v)
    pltpu.sync_copy(dst_hbm.at[wid], dst_v)
    pltpu.sync_copy(uidx_hbm.at[pl.ds(u0, _SRC_PER_W)], uidx_v)

    gls = (gl0, gl1)
    rows = (rows0, rows1)
    gsems = (gsem0, gsem1)
    ssems = (ssem0, ssem1)

    def build(c, gl):
        def body(v, _):
            o = c * _CHUNK + v * 16
            r = uidx_v[pl.ds(o, 16)] + off_v[pl.ds(o, 16)]
            m = jnp.bitwise_and(r, _TR_CW - 1)
            s = (
                jnp.bitwise_and(r, -_TR_CW)
                + jnp.left_shift(jnp.bitwise_and(m, _TR_Q - 1), 2)
                + jnp.right_shift(m, _TR_QS)
            )
            gl[pl.ds(v * 16, 16)] = s
            return 0

        lax.fori_loop(0, _CHUNK // 16, body, 0, unroll=4)

    def start_gather(k):
        cp = pltpu.make_async_copy(
            table_hbm.at[gls[k % 2]], rows[k % 2], gsems[k % 2]
        )
        cp.start()
        return cp

    def start_scatter(k):
        cps = []
        for i in range(_SCAT):
            cp = pltpu.make_async_copy(
                rows[k % 2].at[pl.ds(i * 128, 128)],
                out_hbm.at[dst_v.at[k * _SCAT + i]],
                ssems[k % 2],
            )
            cp.start()
            cps.append(cp)
        return cps

    build(0, gls[0])
    g = start_gather(0)
    build(1, gls[1])
    gn = start_gather(1)
    scat = [(), ()]
    for k in range(_NCHUNK):
        g.wait()
        g = gn
        scat[k % 2] = start_scatter(k)
        nk = k + 2
        if nk < _NCHUNK:
            build(nk, gls[nk % 2])
            for cp in scat[nk % 2]:
                cp.wait()
            gn = start_gather(nk)
    for par in (0, 1):
        for cp in scat[par]:
            cp.wait()


_sc_gather = functools.partial(
    pl.kernel,
    mesh=plsc.VectorSubcoreMesh(core_axis_name="c", subcore_axis_name="s"),
    out_type=jax.ShapeDtypeStruct((_OUT_ROWS, EMB_D), jnp.bfloat16),
    compiler_params=pltpu.CompilerParams(use_tc_tiling_on_sc=False),
    scratch_types=[
        pltpu.VMEM((_SRC_PER_W,), jnp.int32),
        pltpu.VMEM((_NCHUNK * _SCAT, 128), jnp.int32),
        pltpu.VMEM((_SRC_PER_W,), jnp.int32),
        pltpu.VMEM((_CHUNK,), jnp.int32),
        pltpu.VMEM((_CHUNK,), jnp.int32),
        pltpu.VMEM((_CHUNK, EMB_D), jnp.bfloat16),
        pltpu.VMEM((_CHUNK, EMB_D), jnp.bfloat16),
        pltpu.SemaphoreType.DMA,
        pltpu.SemaphoreType.DMA,
        pltpu.SemaphoreType.DMA,
        pltpu.SemaphoreType.DMA,
    ],
)(_sc_gather_body)


_BM = 1024
_KP = _NJ * 128


def _mlp_body(x0, x1, x2, x3, x4, x5, x6, w1_ref, b1_ref, w2_ref, b2_ref, o_ref):
    x6v = x6[...]
    x6v = jnp.concatenate(
        [x6v[:, :64], jnp.zeros((_BM, 64), jnp.bfloat16)], axis=1
    )
    x = jnp.concatenate(
        [x0[...], x1[...], x2[...], x3[...], x4[...], x5[...], x6v], axis=1
    )
    h = jnp.dot(x, w1_ref[...], preferred_element_type=jnp.float32)
    h = jnp.maximum(h + b1_ref[...], 0.0)
    t = jnp.dot(h, w2_ref[...], preferred_element_type=jnp.float32) + b2_ref[...]
    ss = jnp.sum(t * t, axis=-1, keepdims=True)
    denom = jnp.maximum(jnp.sqrt(ss), 1e-12)
    o_ref[...] = t / denom


def _mlp(x7, w1p, b1, w2, b2):
    nb = BATCH // _BM
    xspecs = [
        pl.BlockSpec((_BM, 128), functools.partial(lambda j, i: (j * nb + i, 0), j))
        for j in range(_NJ)
    ]
    return pl.pallas_call(
        _mlp_body,
        grid=(nb,),
        in_specs=xspecs
        + [
            pl.BlockSpec((_KP, HIDDEN), lambda i: (0, 0)),
            pl.BlockSpec((1, HIDDEN), lambda i: (0, 0)),
            pl.BlockSpec((HIDDEN, OUT_D), lambda i: (0, 0)),
            pl.BlockSpec((1, OUT_D), lambda i: (0, 0)),
        ],
        out_specs=pl.BlockSpec((_BM, OUT_D), lambda i: (i, 0)),
        out_shape=jax.ShapeDtypeStruct((BATCH, OUT_D), jnp.float32),
    )(*x7, w1p, b1, w2, b2)


def kernel(user_sparse_indices, tables, W1, b1, W2, b2):
    t128 = _transpose_table(tables.T)
    t_sc = t128.reshape(_TBL_LINES * 4, EMB_D)

    off = jnp.asarray(_OFF_TBL)
    dst = jnp.asarray(_DST_TBL)
    gathered = _sc_gather(off, dst, user_sparse_indices.reshape(-1), t_sc)
    x7 = [gathered.reshape(_NJ * BATCH, 128)] * _NJ

    w1p = jnp.concatenate(
        [W1.astype(jnp.bfloat16),
         jnp.zeros((_KP - F_FIELDS * EMB_D, HIDDEN), jnp.bfloat16)], axis=0
    )
    return _mlp(x7, w1p, b1.reshape(1, HIDDEN), W2, b2.reshape(1, OUT_D))


# speedup vs baseline: 2.5762x; 2.5762x over previous
import functools

import jax
import jax.numpy as jnp
import numpy as np
from jax import lax
from jax.experimental import pallas as pl
from jax.experimental.pallas import tpu as pltpu
from jax.experimental.pallas import tpu_sc as plsc

F_FIELDS = 26
VOCAB = 100000
EMB_D = 32
BATCH = 16384
HIDDEN = 1024
OUT_D = 64

_info = plsc.get_sparse_core_info()
_NC, _NS = _info.num_cores, _info.num_subcores
_NW = _NC * _NS

_NJ = 7
_FH = 13
_SRC_H = _FH * BATCH
_SRC_PER_W = _SRC_H // _NW
_CHUNK = 512
_NCHUNK = _SRC_PER_W // _CHUNK
_SCAT = _CHUNK // 128
_OUT_ROWS_H = 4 * 4 * BATCH

_TR_Q = 4096
_TR_QS = 12
_TR_CW = 4 * _TR_Q
_TR_GRID_H = 80
_TR_B0 = _TR_GRID_H - 1
_TBL_LINES_H = _TR_GRID_H * _TR_Q
_SB_B = 4 * _TR_B0 * _TR_Q


def _tr_body(xt_ref, o_ref):
    x = xt_ref[...]
    z = jnp.concatenate(
        [x[:, u * _TR_Q:(u + 1) * _TR_Q] for u in range(4)], axis=0
    )
    o_ref[...] = jnp.swapaxes(z, 0, 1)


def _transpose_half(tables_t, block0):
    return pl.pallas_call(
        _tr_body,
        grid=(_TR_GRID_H,),
        in_specs=[
            pl.BlockSpec(
                (EMB_D, _TR_CW),
                functools.partial(lambda b0, i: (0, i + b0), block0),
            )
        ],
        out_specs=pl.BlockSpec((_TR_Q, 128), lambda i: (i, 0)),
        out_shape=jax.ShapeDtypeStruct((_TBL_LINES_H, 128), jnp.float32),
    )(tables_t)


def _mk_tables(fbase):
    off = np.empty((_NW, _SRC_PER_W), np.int32)
    dst = np.empty((_NW, _SRC_PER_W), np.int32)
    jshift = 3 if fbase else 0
    for w in range(_NW):
        p = fbase * BATCH + w * _SRC_PER_W + np.arange(_SRC_PER_W)
        f = p // BATCH
        b = p % BATCH
        j = np.minimum(f // 4, _NJ - 1)
        t = f - 4 * j
        off[w] = (f * VOCAB).astype(np.int32)
        dst[w] = ((j - jshift) * (4 * BATCH) + 4 * b + t).astype(np.int32)
    return off, dst.reshape(_NW, _NCHUNK * _SCAT, 128)


_OFF_A, _DST_A = _mk_tables(0)
_OFF_B, _DST_B = _mk_tables(_FH)


def _mk_sc_gather(sbase, ubase):
    def body(off_hbm, dst_hbm, uidx_hbm, table_hbm, out_hbm,
             off_v, dst_v, uidx_v, gl0, gl1, rows0, rows1,
             gsem0, gsem1, ssem0, ssem1):
        wid = lax.axis_index("s") * _NC + lax.axis_index("c")
        u0 = pl.multiple_of(ubase + wid * _SRC_PER_W, 8)
        pltpu.sync_copy(off_hbm.at[wid], off_v)
        pltpu.sync_copy(dst_hbm.at[wid], dst_v)
        pltpu.sync_copy(uidx_hbm.at[pl.ds(u0, _SRC_PER_W)], uidx_v)

        gls = (gl0, gl1)
        rows = (rows0, rows1)
        gsems = (gsem0, gsem1)
        ssems = (ssem0, ssem1)

        def build(c, gl):
            def bd(v, _):
                o = c * _CHUNK + v * 16
                r = uidx_v[pl.ds(o, 16)] + off_v[pl.ds(o, 16)]
                m = jnp.bitwise_and(r, _TR_CW - 1)
                s = (
                    jnp.bitwise_and(r, -_TR_CW)
                    + jnp.left_shift(jnp.bitwise_and(m, _TR_Q - 1), 2)
                    + jnp.right_shift(m, _TR_QS)
                )
                if sbase:
                    s = s - sbase
                gl[pl.ds(v * 16, 16)] = s
                return 0

            lax.fori_loop(0, _CHUNK // 16, bd, 0, unroll=4)

        def start_gather(k):
            cp = pltpu.make_async_copy(
                table_hbm.at[gls[k % 2]], rows[k % 2], gsems[k % 2]
            )
            cp.start()
            return cp

        def start_scatter(k):
            cps = []
            for i in range(_SCAT):
                cp = pltpu.make_async_copy(
                    rows[k % 2].at[pl.ds(i * 128, 128)],
                    out_hbm.at[dst_v.at[k * _SCAT + i]],
                    ssems[k % 2],
                )
                cp.start()
                cps.append(cp)
            return cps

        build(0, gls[0])
        g = start_gather(0)
        build(1, gls[1])
        gn = start_gather(1)
        scat = [(), ()]
        for k in range(_NCHUNK):
            g.wait()
            g = gn
            scat[k % 2] = start_scatter(k)
            nk = k + 2
            if nk < _NCHUNK:
                build(nk, gls[nk % 2])
                for cp in scat[nk % 2]:
                    cp.wait()
                gn = start_gather(nk)
        for par in (0, 1):
            for cp in scat[par]:
                cp.wait()

    return functools.partial(
        pl.kernel,
        mesh=plsc.VectorSubcoreMesh(core_axis_name="c", subcore_axis_name="s"),
        out_type=jax.ShapeDtypeStruct((_OUT_ROWS_H, EMB_D), jnp.float32),
        compiler_params=pltpu.CompilerParams(use_tc_tiling_on_sc=False),
        scratch_types=[
            pltpu.VMEM((_SRC_PER_W,), jnp.int32),
            pltpu.VMEM((_NCHUNK * _SCAT, 128), jnp.int32),
            pltpu.VMEM((_SRC_PER_W,), jnp.int32),
            pltpu.VMEM((_CHUNK,), jnp.int32),
            pltpu.VMEM((_CHUNK,), jnp.int32),
            pltpu.VMEM((_CHUNK, EMB_D), jnp.float32),
            pltpu.VMEM((_CHUNK, EMB_D), jnp.float32),
            pltpu.SemaphoreType.DMA,
            pltpu.SemaphoreType.DMA,
            pltpu.SemaphoreType.DMA,
            pltpu.SemaphoreType.DMA,
        ],
    )(body)


_sc_gather_a = _mk_sc_gather(0, 0)
_sc_gather_b = _mk_sc_gather(_SB_B, _SRC_H)


_BM = 1024
_KP = _NJ * 128


def _mlp_body(xa0, xa1, xa2, xa3, xb3, xb4, xb5, xb6,
              w1_ref, b1_ref, w2_ref, b2_ref, o_ref):
    x3 = jnp.concatenate([xa3[...][:, :32], xb3[...][:, 32:]], axis=1)
    x6 = jnp.concatenate(
        [xb6[...][:, :64], jnp.zeros((_BM, 64), jnp.float32)], axis=1
    )
    x = jnp.concatenate(
        [xa0[...], xa1[...], xa2[...], x3, xb4[...], xb5[...], x6], axis=1
    )
    h = jnp.dot(x, w1_ref[...], preferred_element_type=jnp.float32)
    h = jnp.maximum(h + b1_ref[...], 0.0)
    t = jnp.dot(h, w2_ref[...], preferred_element_type=jnp.float32) + b2_ref[...]
    ss = jnp.sum(t * t, axis=-1, keepdims=True)
    denom = jnp.maximum(jnp.sqrt(ss), 1e-12)
    o_ref[...] = t / denom


def _mlp(xa, xb, w1p, b1, w2, b2):
    nb = BATCH // _BM

    def mk(j):
        return pl.BlockSpec(
            (_BM, 128), functools.partial(lambda jj, i: (jj * nb + i, 0), j)
        )

    xspecs = [mk(0), mk(1), mk(2), mk(3), mk(0), mk(1), mk(2), mk(3)]
    return pl.pallas_call(
        _mlp_body,
        grid=(nb,),
        in_specs=xspecs
        + [
            pl.BlockSpec((_KP, HIDDEN), lambda i: (0, 0)),
            pl.BlockSpec((1, HIDDEN), lambda i: (0, 0)),
            pl.BlockSpec((HIDDEN, OUT_D), lambda i: (0, 0)),
            pl.BlockSpec((1, OUT_D), lambda i: (0, 0)),
        ],
        out_specs=pl.BlockSpec((_BM, OUT_D), lambda i: (i, 0)),
        out_shape=jax.ShapeDtypeStruct((BATCH, OUT_D), jnp.float32),
    )(xa, xa, xa, xa, xb, xb, xb, xb, w1p, b1, w2, b2)


def kernel(user_sparse_indices, tables, W1, b1, W2, b2):
    tables_t = tables.T
    uidx_fm = user_sparse_indices.T.reshape(-1)

    ta = _transpose_half(tables_t, 0)
    ga = _sc_gather_a(
        jnp.asarray(_OFF_A), jnp.asarray(_DST_A), uidx_fm,
        ta.reshape(_TBL_LINES_H * 4, EMB_D),
    )

    tb = _transpose_half(tables_t, _TR_B0)
    gb = _sc_gather_b(
        jnp.asarray(_OFF_B), jnp.asarray(_DST_B), uidx_fm,
        tb.reshape(_TBL_LINES_H * 4, EMB_D),
    )

    xa = ga.reshape(4 * BATCH, 128)
    xb = gb.reshape(4 * BATCH, 128)

    w1p = jnp.concatenate(
        [W1, jnp.zeros((_KP - F_FIELDS * EMB_D, HIDDEN), jnp.float32)], axis=0
    )
    return _mlp(xa, xb, w1p, b1.reshape(1, HIDDEN), W2, b2.reshape(1, OUT_D))
